# Initial kernel scaffold; baseline (speedup 1.0000x reference)
import jax
import jax.numpy as jnp
from jax.experimental import pallas as pl


def kernel(x, w1x1, b1x1, wd1, bd1, wd2, bd2):
    raise NotImplementedError("write your pallas kernel here")


# speedup vs baseline: 1.3436x; 1.3436x over previous
import functools

import jax
import jax.numpy as jnp
from jax.experimental import pallas as pl
from jax.experimental.pallas import tpu as pltpu


def _fused_kernel(x_ref, w_ref, b_ref, wd1t_ref, bd1_ref, wd2_ref, bd2_ref,
                  o_ref, *, img_w, hw, grp, inv_hw):
    x = x_ref[...]
    C = x.shape[0]
    f32 = x.dtype

    lane = jax.lax.broadcasted_iota(jnp.int32, (1, hw), 1)
    wpos = lane % img_w
    em0 = (wpos != img_w - 1).astype(f32)
    em1 = (wpos != 0).astype(f32)
    em2 = (lane < hw - img_w).astype(f32)
    em3 = (lane >= img_w).astype(f32)

    c = jax.lax.broadcasted_iota(jnp.int32, (C, 1), 0)
    m0 = (c < grp).astype(f32)
    m1 = ((c >= grp) & (c < 2 * grp)).astype(f32)
    m2 = ((c >= 2 * grp) & (c < 3 * grp)).astype(f32)
    m3 = ((c >= 3 * grp) & (c < 4 * grp)).astype(f32)
    m4 = (c >= 4 * grp).astype(f32)

    shifted = (m0 * (jnp.roll(x, -1, axis=1) * em0)
               + m1 * (jnp.roll(x, 1, axis=1) * em1)
               + m2 * (jnp.roll(x, -img_w, axis=1) * em2)
               + m3 * (jnp.roll(x, img_w, axis=1) * em3)
               + m4 * x)

    y = jnp.dot(w_ref[...], shifted, preferred_element_type=jnp.float32)
    o_ref[...] = y + b_ref[...]

    pool = jnp.sum(o_ref[...], axis=1, keepdims=True) * inv_hw

    z1 = jnp.maximum(jnp.sum(wd1t_ref[...] * pool, axis=0, keepdims=True)
                     + bd1_ref[...], 0.0)
    scale = jax.nn.sigmoid(jnp.sum(wd2_ref[...] * z1, axis=1, keepdims=True)
                           + bd2_ref[...])

    o_ref[...] = o_ref[...] * scale


def kernel(x, w1x1, b1x1, wd1, bd1, wd2, bd2):
    B, C, H, W = x.shape
    HW = H * W
    Cr = wd1.shape[0]
    g = C // 5
    f32 = jnp.float32

    x3 = x.reshape(B, C, HW).astype(f32)

    out_flat = pl.pallas_call(
        functools.partial(_fused_kernel, img_w=W, hw=HW, grp=g,
                          inv_hw=1.0 / float(HW)),
        out_shape=jax.ShapeDtypeStruct((B, C, HW), f32),
        grid=(B,),
        in_specs=[
            pl.BlockSpec((None, C, HW), lambda b: (b, 0, 0)),
            pl.BlockSpec((C, C), lambda b: (0, 0)),
            pl.BlockSpec((C, 1), lambda b: (0, 0)),
            pl.BlockSpec((C, Cr), lambda b: (0, 0)),
            pl.BlockSpec((1, Cr), lambda b: (0, 0)),
            pl.BlockSpec((C, Cr), lambda b: (0, 0)),
            pl.BlockSpec((C, 1), lambda b: (0, 0)),
        ],
        out_specs=pl.BlockSpec((None, C, HW), lambda b: (b, 0, 0)),
        compiler_params=pltpu.CompilerParams(
            dimension_semantics=("parallel",),
            vmem_limit_bytes=60 * 2 ** 20),
    )(x3, jnp.asarray(w1x1, f32), jnp.asarray(b1x1, f32).reshape(C, 1),
      jnp.asarray(wd1, f32).T, jnp.asarray(bd1, f32).reshape(1, Cr),
      jnp.asarray(wd2, f32), jnp.asarray(bd2, f32).reshape(C, 1))

    return out_flat.reshape(B, C, H, W)


# speedup vs baseline: 1.5088x; 1.1230x over previous
import functools

import jax
import jax.numpy as jnp
from jax.experimental import pallas as pl
from jax.experimental.pallas import tpu as pltpu


def _fused_kernel(x_ref, w_ref, b_ref, wd1t_ref, bd1_ref, wd2_ref, bd2_ref,
                  o_ref, *, img_w, hw, grp, inv_hw):
    bf16 = jnp.bfloat16
    x = x_ref[...].astype(bf16)

    lane = jax.lax.broadcasted_iota(jnp.int32, (1, hw), 1)
    wpos = lane % img_w
    em0 = (wpos != img_w - 1).astype(bf16)
    em1 = (wpos != 0).astype(bf16)
    em2 = (lane < hw - img_w).astype(bf16)
    em3 = (lane >= img_w).astype(bf16)

    shifted = jnp.concatenate([
        jnp.roll(x[0 * grp:1 * grp], -1, axis=1) * em0,
        jnp.roll(x[1 * grp:2 * grp], 1, axis=1) * em1,
        jnp.roll(x[2 * grp:3 * grp], -img_w, axis=1) * em2,
        jnp.roll(x[3 * grp:4 * grp], img_w, axis=1) * em3,
        x[4 * grp:],
    ], axis=0)

    y = jnp.dot(w_ref[...].astype(bf16), shifted,
                preferred_element_type=jnp.float32) + b_ref[...]

    pool = jnp.sum(y, axis=1, keepdims=True) * inv_hw

    z1 = jnp.maximum(jnp.sum(wd1t_ref[...] * pool, axis=0, keepdims=True)
                     + bd1_ref[...], 0.0)
    scale = jax.nn.sigmoid(jnp.sum(wd2_ref[...] * z1, axis=1, keepdims=True)
                           + bd2_ref[...])

    o_ref[...] = y * scale


def kernel(x, w1x1, b1x1, wd1, bd1, wd2, bd2):
    B, C, H, W = x.shape
    HW = H * W
    Cr = wd1.shape[0]
    g = C // 5
    f32 = jnp.float32

    x3 = x.reshape(B, C, HW).astype(f32)

    out_flat = pl.pallas_call(
        functools.partial(_fused_kernel, img_w=W, hw=HW, grp=g,
                          inv_hw=1.0 / float(HW)),
        out_shape=jax.ShapeDtypeStruct((B, C, HW), f32),
        grid=(B,),
        in_specs=[
            pl.BlockSpec((None, C, HW), lambda b: (b, 0, 0)),
            pl.BlockSpec((C, C), lambda b: (0, 0)),
            pl.BlockSpec((C, 1), lambda b: (0, 0)),
            pl.BlockSpec((C, Cr), lambda b: (0, 0)),
            pl.BlockSpec((1, Cr), lambda b: (0, 0)),
            pl.BlockSpec((C, Cr), lambda b: (0, 0)),
            pl.BlockSpec((C, 1), lambda b: (0, 0)),
        ],
        out_specs=pl.BlockSpec((None, C, HW), lambda b: (b, 0, 0)),
        compiler_params=pltpu.CompilerParams(
            dimension_semantics=("parallel",),
            vmem_limit_bytes=60 * 2 ** 20),
    )(x3, jnp.asarray(w1x1, f32), jnp.asarray(b1x1, f32).reshape(C, 1),
      jnp.asarray(wd1, f32).T, jnp.asarray(bd1, f32).reshape(1, Cr),
      jnp.asarray(wd2, f32), jnp.asarray(bd2, f32).reshape(C, 1))

    return out_flat.reshape(B, C, H, W)


# speedup vs baseline: 4.4742x; 2.9654x over previous
import functools

import jax
import jax.numpy as jnp
from jax.experimental import pallas as pl
from jax.experimental.pallas import tpu as pltpu


def _fused_kernel(x_ref, w_ref, b_ref, wd1t_ref, bd1_ref, wd2_ref, bd2_ref,
                  o_ref, *, img_w, hw, grp, inv_hw):
    C = x_ref.shape[0]
    bf16 = jnp.bfloat16
    x = x_ref[...].reshape(C, hw).astype(bf16)

    lane = jax.lax.broadcasted_iota(jnp.int32, (1, hw), 1)
    wpos = lane % img_w
    em0 = (wpos != img_w - 1).astype(bf16)
    em1 = (wpos != 0).astype(bf16)
    em2 = (lane < hw - img_w).astype(bf16)
    em3 = (lane >= img_w).astype(bf16)

    shifted = jnp.concatenate([
        jnp.roll(x[0 * grp:1 * grp], -1, axis=1) * em0,
        jnp.roll(x[1 * grp:2 * grp], 1, axis=1) * em1,
        jnp.roll(x[2 * grp:3 * grp], -img_w, axis=1) * em2,
        jnp.roll(x[3 * grp:4 * grp], img_w, axis=1) * em3,
        x[4 * grp:],
    ], axis=0)

    y = jnp.dot(w_ref[...].astype(bf16), shifted,
                preferred_element_type=jnp.float32) + b_ref[...]

    pool = jnp.sum(y, axis=1, keepdims=True) * inv_hw

    z1 = jnp.maximum(jnp.sum(wd1t_ref[...] * pool, axis=0, keepdims=True)
                     + bd1_ref[...], 0.0)
    scale = jax.nn.sigmoid(jnp.sum(wd2_ref[...] * z1, axis=1, keepdims=True)
                           + bd2_ref[...])

    o_ref[...] = (y * scale).reshape(o_ref.shape)


def kernel(x, w1x1, b1x1, wd1, bd1, wd2, bd2):
    B, C, H, W = x.shape
    HW = H * W
    Cr = wd1.shape[0]
    g = C // 5
    f32 = jnp.float32

    out = pl.pallas_call(
        functools.partial(_fused_kernel, img_w=W, hw=HW, grp=g,
                          inv_hw=1.0 / float(HW)),
        out_shape=jax.ShapeDtypeStruct((B, C, H, W), f32),
        grid=(B,),
        in_specs=[
            pl.BlockSpec((None, C, H, W), lambda b: (b, 0, 0, 0)),
            pl.BlockSpec((C, C), lambda b: (0, 0)),
            pl.BlockSpec((C, 1), lambda b: (0, 0)),
            pl.BlockSpec((C, Cr), lambda b: (0, 0)),
            pl.BlockSpec((1, Cr), lambda b: (0, 0)),
            pl.BlockSpec((C, Cr), lambda b: (0, 0)),
            pl.BlockSpec((C, 1), lambda b: (0, 0)),
        ],
        out_specs=pl.BlockSpec((None, C, H, W), lambda b: (b, 0, 0, 0)),
        compiler_params=pltpu.CompilerParams(
            dimension_semantics=("parallel",),
            vmem_limit_bytes=60 * 2 ** 20),
    )(x.astype(f32), jnp.asarray(w1x1, f32), jnp.asarray(b1x1, f32).reshape(C, 1),
      jnp.asarray(wd1, f32).T, jnp.asarray(bd1, f32).reshape(1, Cr),
      jnp.asarray(wd2, f32), jnp.asarray(bd2, f32).reshape(C, 1))

    return out


# speedup vs baseline: 4.7822x; 1.0688x over previous
import functools

import jax
import jax.numpy as jnp
from jax.experimental import pallas as pl
from jax.experimental.pallas import tpu as pltpu


def _fused_kernel(x_ref, w_ref, b_ref, wd1t_ref, bd1_ref, wd2_ref, bd2_ref,
                  o_ref, *, img_w, hw, grp, inv_hw):
    C = x_ref.shape[0]
    bf16 = jnp.bfloat16
    x = x_ref[...].astype(bf16).reshape(C, hw)

    lane = jax.lax.broadcasted_iota(jnp.int32, (1, hw), 1)
    wpos = lane % img_w
    em0 = (wpos != img_w - 1).astype(bf16)
    em1 = (wpos != 0).astype(bf16)
    em2 = (lane < hw - img_w).astype(bf16)
    em3 = (lane >= img_w).astype(bf16)

    shifted = jnp.concatenate([
        jnp.roll(x[0 * grp:1 * grp], -1, axis=1) * em0,
        jnp.roll(x[1 * grp:2 * grp], 1, axis=1) * em1,
        jnp.roll(x[2 * grp:3 * grp], -img_w, axis=1) * em2,
        jnp.roll(x[3 * grp:4 * grp], img_w, axis=1) * em3,
        x[4 * grp:],
    ], axis=0)

    y = jnp.dot(w_ref[...].astype(bf16), shifted,
                preferred_element_type=jnp.float32) + b_ref[...]

    pool = jnp.sum(y, axis=1, keepdims=True) * inv_hw

    z1 = jnp.maximum(jnp.sum(wd1t_ref[...] * pool, axis=0, keepdims=True)
                     + bd1_ref[...], 0.0)
    scale = jax.nn.sigmoid(jnp.sum(wd2_ref[...] * z1, axis=1, keepdims=True)
                           + bd2_ref[...])

    o_ref[...] = (y * scale).reshape(o_ref.shape)


def kernel(x, w1x1, b1x1, wd1, bd1, wd2, bd2):
    B, C, H, W = x.shape
    HW = H * W
    Cr = wd1.shape[0]
    g = C // 5
    f32 = jnp.float32

    out = pl.pallas_call(
        functools.partial(_fused_kernel, img_w=W, hw=HW, grp=g,
                          inv_hw=1.0 / float(HW)),
        out_shape=jax.ShapeDtypeStruct((B, C, H, W), f32),
        grid=(B,),
        in_specs=[
            pl.BlockSpec((None, C, H, W), lambda b: (b, 0, 0, 0)),
            pl.BlockSpec((C, C), lambda b: (0, 0)),
            pl.BlockSpec((C, 1), lambda b: (0, 0)),
            pl.BlockSpec((C, Cr), lambda b: (0, 0)),
            pl.BlockSpec((1, Cr), lambda b: (0, 0)),
            pl.BlockSpec((C, Cr), lambda b: (0, 0)),
            pl.BlockSpec((C, 1), lambda b: (0, 0)),
        ],
        out_specs=pl.BlockSpec((None, C, H, W), lambda b: (b, 0, 0, 0)),
        compiler_params=pltpu.CompilerParams(
            dimension_semantics=("parallel",),
            vmem_limit_bytes=60 * 2 ** 20),
    )(x.astype(f32), jnp.asarray(w1x1, f32), jnp.asarray(b1x1, f32).reshape(C, 1),
      jnp.asarray(wd1, f32).T, jnp.asarray(bd1, f32).reshape(1, Cr),
      jnp.asarray(wd2, f32), jnp.asarray(bd2, f32).reshape(C, 1))

    return out
